# baseline (speedup 1.0000x reference)
import functools

import jax
import jax.numpy as jnp
from jax.experimental import pallas as pl
from jax.experimental.pallas import tpu as pltpu


_NU = 2000
_NP = 8000
_NV = 10000
_NB = 5000
_NBP = 5120
_D = 128

_BM_T = 1024
_BK_T = 2000
_BM = 2000
_BK = 1024


def _mm_t_kernel(a_ref, x_ref, s_ref, o_ref, acc_ref, *, nk):
    k = pl.program_id(1)

    @pl.when(k == 0)
    def _():
        acc_ref[...] = jnp.zeros_like(acc_ref)

    w = (s_ref[...] * x_ref[...]).astype(jnp.bfloat16)
    acc_ref[...] += jax.lax.dot_general(
        a_ref[...], w, (((0,), (0,)), ((), ())),
        preferred_element_type=jnp.float32)

    @pl.when(k == nk - 1)
    def _():
        o_ref[...] = acc_ref[...]


def _mm_kernel(a_ref, z_ref, de_ref, dv_ref, o_ref, acc_ref, *, nk):
    k = pl.program_id(1)

    @pl.when(k == 0)
    def _():
        acc_ref[...] = jnp.zeros_like(acc_ref)

    v = (de_ref[...] * z_ref[...]).astype(jnp.bfloat16)
    acc_ref[...] += jnp.dot(a_ref[...], v, preferred_element_type=jnp.float32)

    @pl.when(k == nk - 1)
    def _():
        o_ref[...] = dv_ref[...] * acc_ref[...]


def _mm_final_kernel(a_ref, z_ref, de_ref, dv_ref, x0_ref, x1_ref, o_ref,
                     acc_ref, *, nk):
    k = pl.program_id(1)

    @pl.when(k == 0)
    def _():
        acc_ref[...] = jnp.zeros_like(acc_ref)

    v = (de_ref[...] * z_ref[...]).astype(jnp.bfloat16)
    acc_ref[...] += jnp.dot(a_ref[...], v, preferred_element_type=jnp.float32)

    @pl.when(k == nk - 1)
    def _():
        o_ref[...] = (x0_ref[...] + x1_ref[...]
                      + dv_ref[...] * acc_ref[...]) * (1.0 / 3.0)


def _mm_t(a16, x, s, *, interpret=False):
    nm, nk = _NBP // _BM_T, _NV // _BK_T
    return pl.pallas_call(
        functools.partial(_mm_t_kernel, nk=nk),
        grid=(nm, nk),
        in_specs=[
            pl.BlockSpec((_BK_T, _BM_T), lambda m, k: (k, m)),
            pl.BlockSpec((_BK_T, _D), lambda m, k: (k, 0)),
            pl.BlockSpec((_BK_T, 1), lambda m, k: (k, 0)),
        ],
        out_specs=pl.BlockSpec((_BM_T, _D), lambda m, k: (m, 0)),
        out_shape=jax.ShapeDtypeStruct((_NBP, _D), jnp.float32),
        scratch_shapes=[pltpu.VMEM((_BM_T, _D), jnp.float32)],
        interpret=interpret,
    )(a16, x, s)


def _mm(a16, z, de, dv, *, interpret=False):
    nm, nk = _NV // _BM, _NBP // _BK
    return pl.pallas_call(
        functools.partial(_mm_kernel, nk=nk),
        grid=(nm, nk),
        in_specs=[
            pl.BlockSpec((_BM, _BK), lambda m, k: (m, k)),
            pl.BlockSpec((_BK, _D), lambda m, k: (k, 0)),
            pl.BlockSpec((_BK, 1), lambda m, k: (k, 0)),
            pl.BlockSpec((_BM, 1), lambda m, k: (m, 0)),
        ],
        out_specs=pl.BlockSpec((_BM, _D), lambda m, k: (m, 0)),
        out_shape=jax.ShapeDtypeStruct((_NV, _D), jnp.float32),
        scratch_shapes=[pltpu.VMEM((_BM, _D), jnp.float32)],
        interpret=interpret,
    )(a16, z, de, dv)


def _mm_final(a16, z, de, dv, x0, x1, *, interpret=False):
    nm, nk = _NV // _BM, _NBP // _BK
    return pl.pallas_call(
        functools.partial(_mm_final_kernel, nk=nk),
        grid=(nm, nk),
        in_specs=[
            pl.BlockSpec((_BM, _BK), lambda m, k: (m, k)),
            pl.BlockSpec((_BK, _D), lambda m, k: (k, 0)),
            pl.BlockSpec((_BK, 1), lambda m, k: (k, 0)),
            pl.BlockSpec((_BM, 1), lambda m, k: (m, 0)),
            pl.BlockSpec((_BM, _D), lambda m, k: (m, 0)),
            pl.BlockSpec((_BM, _D), lambda m, k: (m, 0)),
        ],
        out_specs=pl.BlockSpec((_BM, _D), lambda m, k: (m, 0)),
        out_shape=jax.ShapeDtypeStruct((_NV, _D), jnp.float32),
        scratch_shapes=[pltpu.VMEM((_BM, _D), jnp.float32)],
        interpret=interpret,
    )(a16, z, de, dv, x0, x1)


def _run(users_embedding, product_embedding, adj_matrix, degreeV_matrix,
         degreeE_matrix, gate_user, gate_product, interpret=False):
    num_users = users_embedding.shape[0]
    x0 = jnp.concatenate([users_embedding, product_embedding], axis=0)
    a16 = jnp.pad(adj_matrix.astype(jnp.bfloat16), ((0, 0), (0, _NBP - _NB)))
    dv = degreeV_matrix[:, None]
    de = jnp.pad(degreeE_matrix, (0, _NBP - _NB))[:, None]
    gates = jnp.concatenate([
        jnp.broadcast_to(gate_user, (num_users, 1)),
        jnp.broadcast_to(gate_product, (_NV - num_users, 1)),
    ])
    s = dv * gates

    z1 = _mm_t(a16, x0, s, interpret=interpret)
    x1 = _mm(a16, z1, de, dv, interpret=interpret)
    z2 = _mm_t(a16, x1, s, interpret=interpret)
    out = _mm_final(a16, z2, de, dv, x0, x1, interpret=interpret)
    return out[:num_users], out[num_users:]


def kernel(users_embedding, product_embedding, adj_matrix, degreeV_matrix,
           degreeE_matrix, gate_user, gate_product):
    return _run(users_embedding, product_embedding, adj_matrix,
                degreeV_matrix, degreeE_matrix, gate_user, gate_product)


# speedup vs baseline: 1.7288x; 1.7288x over previous
import functools

import jax
import jax.numpy as jnp
from jax.experimental import pallas as pl
from jax.experimental.pallas import tpu as pltpu


_NV = 10000
_NB = 5000
_D = 128

_BK1 = 400
_BM2 = 1000
_BM4 = 1000


def _p1_kernel(a_ref, x_ref, s_ref, a16_ref, z_ref, acc_ref, *, nk):
    k = pl.program_id(0)

    @pl.when(k == 0)
    def _():
        acc_ref[...] = jnp.zeros_like(acc_ref)

    a16 = a_ref[...].astype(jnp.bfloat16)
    a16_ref[...] = a16
    w = (s_ref[...] * x_ref[...]).astype(jnp.bfloat16)
    acc_ref[...] += jax.lax.dot_general(
        a16, w, (((0,), (0,)), ((), ())), preferred_element_type=jnp.float32)

    @pl.when(k == nk - 1)
    def _():
        z_ref[...] = acc_ref[...]


def _p23_kernel(a16_ref, z1_ref, de_ref, s_ref, dv_ref, x1_ref, z2_ref,
                v16_ref, acc_ref, *, nk):
    r = pl.program_id(0)

    @pl.when(r == 0)
    def _():
        acc_ref[...] = jnp.zeros_like(acc_ref)
        v16_ref[...] = (de_ref[...] * z1_ref[...]).astype(jnp.bfloat16)

    a16 = a16_ref[...]
    x1 = dv_ref[...] * jnp.dot(a16, v16_ref[...],
                               preferred_element_type=jnp.float32)
    x1_ref[...] = x1
    w = (s_ref[...] * x1).astype(jnp.bfloat16)
    acc_ref[...] += jax.lax.dot_general(
        a16, w, (((0,), (0,)), ((), ())), preferred_element_type=jnp.float32)

    @pl.when(r == nk - 1)
    def _():
        z2_ref[...] = acc_ref[...]


def _p4_kernel(a16_ref, z2_ref, de_ref, dv_ref, x0_ref, x1_ref, o_ref,
               v16_ref):
    m = pl.program_id(0)

    @pl.when(m == 0)
    def _():
        v16_ref[...] = (de_ref[...] * z2_ref[...]).astype(jnp.bfloat16)

    x2 = dv_ref[...] * jnp.dot(a16_ref[...], v16_ref[...],
                               preferred_element_type=jnp.float32)
    o_ref[...] = (x0_ref[...] + x1_ref[...] + x2) * (1.0 / 3.0)


def _p1(a, x0, s, *, interpret=False):
    nk = _NV // _BK1
    return pl.pallas_call(
        functools.partial(_p1_kernel, nk=nk),
        grid=(nk,),
        in_specs=[
            pl.BlockSpec((_BK1, _NB), lambda k: (k, 0)),
            pl.BlockSpec((_BK1, _D), lambda k: (k, 0)),
            pl.BlockSpec((_BK1, 1), lambda k: (k, 0)),
        ],
        out_specs=[
            pl.BlockSpec((_BK1, _NB), lambda k: (k, 0)),
            pl.BlockSpec((_NB, _D), lambda k: (0, 0)),
        ],
        out_shape=[
            jax.ShapeDtypeStruct((_NV, _NB), jnp.bfloat16),
            jax.ShapeDtypeStruct((_NB, _D), jnp.float32),
        ],
        scratch_shapes=[pltpu.VMEM((_NB, _D), jnp.float32)],
        interpret=interpret,
    )(a, x0, s)


def _p23(a16, z1, de, s, dv, *, interpret=False):
    nk = _NV // _BM2
    return pl.pallas_call(
        functools.partial(_p23_kernel, nk=nk),
        grid=(nk,),
        in_specs=[
            pl.BlockSpec((_BM2, _NB), lambda r: (r, 0)),
            pl.BlockSpec((_NB, _D), lambda r: (0, 0)),
            pl.BlockSpec((_NB, 1), lambda r: (0, 0)),
            pl.BlockSpec((_BM2, 1), lambda r: (r, 0)),
            pl.BlockSpec((_BM2, 1), lambda r: (r, 0)),
        ],
        out_specs=[
            pl.BlockSpec((_BM2, _D), lambda r: (r, 0)),
            pl.BlockSpec((_NB, _D), lambda r: (0, 0)),
        ],
        out_shape=[
            jax.ShapeDtypeStruct((_NV, _D), jnp.float32),
            jax.ShapeDtypeStruct((_NB, _D), jnp.float32),
        ],
        scratch_shapes=[
            pltpu.VMEM((_NB, _D), jnp.bfloat16),
            pltpu.VMEM((_NB, _D), jnp.float32),
        ],
        interpret=interpret,
    )(a16, z1, de, s, dv)


def _p4(a16, z2, de, dv, x0, x1, *, interpret=False):
    nm = _NV // _BM4
    return pl.pallas_call(
        _p4_kernel,
        grid=(nm,),
        in_specs=[
            pl.BlockSpec((_BM4, _NB), lambda m: (m, 0)),
            pl.BlockSpec((_NB, _D), lambda m: (0, 0)),
            pl.BlockSpec((_NB, 1), lambda m: (0, 0)),
            pl.BlockSpec((_BM4, 1), lambda m: (m, 0)),
            pl.BlockSpec((_BM4, _D), lambda m: (m, 0)),
            pl.BlockSpec((_BM4, _D), lambda m: (m, 0)),
        ],
        out_specs=pl.BlockSpec((_BM4, _D), lambda m: (m, 0)),
        out_shape=jax.ShapeDtypeStruct((_NV, _D), jnp.float32),
        scratch_shapes=[pltpu.VMEM((_NB, _D), jnp.bfloat16)],
        interpret=interpret,
    )(a16, z2, de, dv, x0, x1)


def _run(users_embedding, product_embedding, adj_matrix, degreeV_matrix,
         degreeE_matrix, gate_user, gate_product, interpret=False):
    num_users = users_embedding.shape[0]
    x0 = jnp.concatenate([users_embedding, product_embedding], axis=0)
    dv = degreeV_matrix[:, None]
    de = degreeE_matrix[:, None]
    gates = jnp.concatenate([
        jnp.broadcast_to(gate_user, (num_users, 1)),
        jnp.broadcast_to(gate_product, (_NV - num_users, 1)),
    ])
    s = dv * gates

    a16, z1 = _p1(adj_matrix, x0, s, interpret=interpret)
    x1, z2 = _p23(a16, z1, de, s, dv, interpret=interpret)
    out = _p4(a16, z2, de, dv, x0, x1, interpret=interpret)
    return out[:num_users], out[num_users:]


def kernel(users_embedding, product_embedding, adj_matrix, degreeV_matrix,
           degreeE_matrix, gate_user, gate_product):
    return _run(users_embedding, product_embedding, adj_matrix,
                degreeV_matrix, degreeE_matrix, gate_user, gate_product)


# speedup vs baseline: 1.8838x; 1.0897x over previous
import functools

import jax
import jax.numpy as jnp
from jax.experimental import pallas as pl
from jax.experimental.pallas import tpu as pltpu


_NV = 10000
_NB = 5000
_NBP = 5120
_D = 128

_BK1 = 400
_BM = 1000


def _pass1_kernel(a_ref, x0_ref, s_ref, de_ref, a16_ref, z1_ref, acc_ref,
                  *, nk):
    k = pl.program_id(0)

    @pl.when(k == 0)
    def _():
        acc_ref[...] = jnp.zeros_like(acc_ref)

    a16 = a_ref[...].astype(jnp.bfloat16)
    a16p = jnp.concatenate(
        [a16, jnp.zeros((a16.shape[0], _NBP - _NB), jnp.bfloat16)], axis=1)
    a16_ref[...] = a16p

    w = (s_ref[...] * x0_ref[...]).astype(jnp.bfloat16)
    acc_ref[...] += jax.lax.dot(w.T, a16p,
                                preferred_element_type=jnp.float32)

    @pl.when(k == nk - 1)
    def _():
        z1_ref[...] = (acc_ref[...] * de_ref[...]).astype(jnp.bfloat16).T


def _pass1(a, x0, s, de_row, *, interpret=False):
    nk = _NV // _BK1
    return pl.pallas_call(
        functools.partial(_pass1_kernel, nk=nk),
        grid=(nk,),
        in_specs=[
            pl.BlockSpec((_BK1, _NB), lambda k: (k, 0)),
            pl.BlockSpec((_BK1, _D), lambda k: (k, 0)),
            pl.BlockSpec((_BK1, 1), lambda k: (k, 0)),
            pl.BlockSpec((1, _NBP), lambda k: (0, 0)),
        ],
        out_specs=[
            pl.BlockSpec((_BK1, _NBP), lambda k: (k, 0)),
            pl.BlockSpec((_NBP, _D), lambda k: (0, 0)),
        ],
        out_shape=[
            jax.ShapeDtypeStruct((_NV, _NBP), jnp.bfloat16),
            jax.ShapeDtypeStruct((_NBP, _D), jnp.bfloat16),
        ],
        scratch_shapes=[pltpu.VMEM((_D, _NBP), jnp.float32)],
        interpret=interpret,
    )(a, x0, s, de_row)


def _pass2_kernel(a16_ref, z1_ref, s_ref, dv_ref, de_ref, x1_ref, z2_ref,
                  acc_ref, *, nm):
    m = pl.program_id(0)

    @pl.when(m == 0)
    def _():
        acc_ref[...] = jnp.zeros_like(acc_ref)

    x1 = dv_ref[...] * jax.lax.dot(a16_ref[...], z1_ref[...],
                                   preferred_element_type=jnp.float32)
    x1_ref[...] = x1

    w = (s_ref[...] * x1).astype(jnp.bfloat16)
    acc_ref[...] += jax.lax.dot(w.T, a16_ref[...],
                                preferred_element_type=jnp.float32)

    @pl.when(m == nm - 1)
    def _():
        z2_ref[...] = (acc_ref[...] * de_ref[...]).astype(jnp.bfloat16).T


def _pass2(a16, z1, s, dv, de_row, *, interpret=False):
    nm = _NV // _BM
    return pl.pallas_call(
        functools.partial(_pass2_kernel, nm=nm),
        grid=(nm,),
        in_specs=[
            pl.BlockSpec((_BM, _NBP), lambda m: (m, 0)),
            pl.BlockSpec((_NBP, _D), lambda m: (0, 0)),
            pl.BlockSpec((_BM, 1), lambda m: (m, 0)),
            pl.BlockSpec((_BM, 1), lambda m: (m, 0)),
            pl.BlockSpec((1, _NBP), lambda m: (0, 0)),
        ],
        out_specs=[
            pl.BlockSpec((_BM, _D), lambda m: (m, 0)),
            pl.BlockSpec((_NBP, _D), lambda m: (0, 0)),
        ],
        out_shape=[
            jax.ShapeDtypeStruct((_NV, _D), jnp.float32),
            jax.ShapeDtypeStruct((_NBP, _D), jnp.bfloat16),
        ],
        scratch_shapes=[pltpu.VMEM((_D, _NBP), jnp.float32)],
        interpret=interpret,
    )(a16, z1, s, dv, de_row)


def _pass3_kernel(a16_ref, z2_ref, dv_ref, x0_ref, x1_ref, o_ref):
    x2 = dv_ref[...] * jax.lax.dot(a16_ref[...], z2_ref[...],
                                   preferred_element_type=jnp.float32)
    o_ref[...] = (x0_ref[...] + x1_ref[...] + x2) * (1.0 / 3.0)


def _pass3(a16, z2, dv, x0, x1, *, interpret=False):
    nm = _NV // _BM
    return pl.pallas_call(
        _pass3_kernel,
        grid=(nm,),
        in_specs=[
            pl.BlockSpec((_BM, _NBP), lambda m: (m, 0)),
            pl.BlockSpec((_NBP, _D), lambda m: (0, 0)),
            pl.BlockSpec((_BM, 1), lambda m: (m, 0)),
            pl.BlockSpec((_BM, _D), lambda m: (m, 0)),
            pl.BlockSpec((_BM, _D), lambda m: (m, 0)),
        ],
        out_specs=pl.BlockSpec((_BM, _D), lambda m: (m, 0)),
        out_shape=jax.ShapeDtypeStruct((_NV, _D), jnp.float32),
        interpret=interpret,
    )(a16, z2, dv, x0, x1)


def _run(users_embedding, product_embedding, adj_matrix, degreeV_matrix,
         degreeE_matrix, gate_user, gate_product, interpret=False):
    num_users = users_embedding.shape[0]
    x0 = jnp.concatenate([users_embedding, product_embedding], axis=0)
    dv = degreeV_matrix[:, None]
    de_row = jnp.pad(degreeE_matrix, (0, _NBP - _NB))[None, :]
    gates = jnp.concatenate([
        jnp.broadcast_to(gate_user, (num_users, 1)),
        jnp.broadcast_to(gate_product, (_NV - num_users, 1)),
    ])
    s = dv * gates

    a16, z1 = _pass1(adj_matrix, x0, s, de_row, interpret=interpret)
    x1, z2 = _pass2(a16, z1, s, dv, de_row, interpret=interpret)
    out = _pass3(a16, z2, dv, x0, x1, interpret=interpret)
    return out[:num_users], out[num_users:]


def kernel(users_embedding, product_embedding, adj_matrix, degreeV_matrix,
           degreeE_matrix, gate_user, gate_product):
    return _run(users_embedding, product_embedding, adj_matrix,
                degreeV_matrix, degreeE_matrix, gate_user, gate_product)
